# baseline (speedup 1.0000x reference)
import functools

import jax
import jax.numpy as jnp
from jax import lax
from jax.experimental import pallas as pl
from jax.experimental.pallas import tpu as pltpu
from jax.experimental.pallas import tpu_sc as plsc

_NC = 2
_NS = 16
_NW = _NC * _NS
_L = 16
_SG = 10


def kernel(token_ids, W):
    B, S = token_ids.shape
    dim = W.shape[1]
    TB = B // 128
    tb_per_w = TB // _NW
    n_sg = S // _SG
    n_ct = dim // 8

    ids_t = jnp.transpose(token_ids)

    w_lin = jax.lax.optimization_barrier(W.reshape(W.shape[0] // 4, dim * 4))
    w_kern = w_lin.reshape(W.shape)

    mesh = plsc.VectorSubcoreMesh(core_axis_name="c", subcore_axis_name="s")

    @functools.partial(
        pl.kernel,
        mesh=mesh,
        out_type=jax.ShapeDtypeStruct((S, n_ct, TB, 8, 128), W.dtype),
        scratch_types=[
            pltpu.VMEM((S, 128), jnp.int32),
            pltpu.VMEM((_SG, 128, dim), W.dtype),
            pltpu.VMEM((_SG, n_ct, 8, 128), W.dtype),
            pltpu.SemaphoreType.DMA,
            pltpu.SemaphoreType.DMA,
        ],
        compiler_params=pltpu.CompilerParams(
            use_tc_tiling_on_sc=False, needs_layout_passes=False
        ),
    )
    def gather_kernel(table_hbm, idx_hbm, out_hbm, idx_v, g_v, o_v, gsem, osem):
        wid = lax.axis_index("s") * _NC + lax.axis_index("c")

        iotas = [
            jax.lax.broadcasted_iota(jnp.int32, (_L,), 0) + bl0 * _L
            for bl0 in range(128 // _L)
        ]

        @pl.loop(0, tb_per_w)
        def _(t):
            tb = wid * tb_per_w + t
            pltpu.sync_copy(idx_hbm.at[:, pl.ds(tb * 128, 128)], idx_v)

            @pl.loop(0, n_sg)
            def _(sg):
                s0 = sg * _SG
                for j in range(_SG):
                    pltpu.async_copy(
                        table_hbm.at[idx_v.at[s0 + j]], g_v.at[j], gsem
                    )
                for j in range(_SG):
                    pltpu.make_async_copy(
                        table_hbm.at[idx_v.at[s0 + j]], g_v.at[j], gsem
                    ).wait()
                for j in range(_SG):
                    jv = jnp.full((_L,), j, jnp.int32)
                    for c in range(dim):
                        cv = jnp.full((_L,), c, jnp.int32)
                        for bl0 in range(128 // _L):
                            vals = plsc.load_gather(
                                g_v, [jv, iotas[bl0], cv]
                            )
                            o_v[j, c // 8, c % 8, pl.ds(bl0 * _L, _L)] = vals
                pltpu.async_copy(
                    o_v, out_hbm.at[pl.ds(s0, _SG), :, tb], osem
                ).wait()

    out_a = gather_kernel(w_kern, ids_t)
    t = jnp.transpose(out_a, (2, 4, 0, 1, 3))
    return t.reshape(B, S, dim)


# speedup vs baseline: 1.5063x; 1.5063x over previous
import functools

import jax
import jax.numpy as jnp
from jax import lax
from jax.experimental import pallas as pl
from jax.experimental.pallas import tpu as pltpu
from jax.experimental.pallas import tpu_sc as plsc

_NC = 2
_NS = 16
_NW = _NC * _NS
_L = 16
_SG = 10
_OP = 129


def kernel(token_ids, W):
    B, S = token_ids.shape
    dim = W.shape[1]
    TB = B // 128
    tb_per_w = TB // _NW
    n_sg = S // _SG
    n_ct = dim // 8

    ids_t = jnp.transpose(token_ids)

    mesh = plsc.VectorSubcoreMesh(core_axis_name="c", subcore_axis_name="s")

    @functools.partial(
        pl.kernel,
        mesh=mesh,
        out_type=jax.ShapeDtypeStruct((S, n_ct, TB, 8, 128), W.dtype),
        scratch_types=[
            pltpu.VMEM((S, 128), jnp.int32),
            pltpu.VMEM((_SG, 128, 32), W.dtype),
            pltpu.VMEM((_SG, 32, _OP), W.dtype),
            pltpu.SemaphoreType.DMA,
            pltpu.SemaphoreType.DMA,
        ],
        compiler_params=pltpu.CompilerParams(
            use_tc_tiling_on_sc=False, needs_layout_passes=False
        ),
    )
    def gather_kernel(
        table_hbm, idx_hbm, out_hbm, idx_v, g_v, o_v, gsem, osem
    ):
        wid = lax.axis_index("s") * _NC + lax.axis_index("c")

        iota = jax.lax.broadcasted_iota(jnp.int32, (_L,), 0)
        cvecs = [iota + k * _L for k in range(dim // _L)]

        @pl.loop(0, tb_per_w)
        def _(t):
            tb = wid * tb_per_w + t
            pltpu.sync_copy(idx_hbm.at[:, pl.ds(tb * 128, 128)], idx_v)

            @pl.loop(0, n_sg)
            def _(sg):
                s0 = sg * _SG
                for j in range(_SG):
                    pltpu.async_copy(
                        table_hbm.at[idx_v.at[s0 + j]], g_v.at[j], gsem
                    )
                for j in range(_SG):
                    pltpu.make_async_copy(
                        table_hbm.at[idx_v.at[s0 + j]], g_v.at[j], gsem
                    ).wait()
                for j in range(_SG):
                    jv = jnp.full((_L,), j, jnp.int32)

                    @pl.loop(0, 128 // _L)
                    def _(blb):
                        for bi in range(_L):
                            bl = blb * _L + bi
                            blv = jnp.full((_L,), bl, jnp.int32)
                            for k in range(dim // _L):
                                vals = g_v[j, bl, pl.ds(k * _L, _L)]
                                plsc.store_scatter(
                                    o_v, [jv, cvecs[k], blv], vals
                                )
                    for tc in range(n_ct):
                        pltpu.async_copy(
                            o_v.at[j, pl.ds(tc * 8, 8), pl.ds(0, 128)],
                            out_hbm.at[s0 + j, tc, tb],
                            osem,
                        )
                for j in range(_SG):
                    for tc in range(n_ct):
                        pltpu.make_async_copy(
                            o_v.at[j, pl.ds(tc * 8, 8), pl.ds(0, 128)],
                            out_hbm.at[s0 + j, tc, tb],
                            osem,
                        ).wait()

    out_a = gather_kernel(W, ids_t)
    t = jnp.transpose(out_a, (2, 4, 0, 1, 3))
    return t.reshape(B, S, dim)


# speedup vs baseline: 1.5473x; 1.0272x over previous
import functools

import jax
import jax.numpy as jnp
from jax import lax
from jax.experimental import pallas as pl
from jax.experimental.pallas import tpu as pltpu
from jax.experimental.pallas import tpu_sc as plsc

_NC = 2
_NS = 16
_NW = _NC * _NS
_L = 16
_SG = 10
_OP = 129


def kernel(token_ids, W):
    B, S = token_ids.shape
    dim = W.shape[1]
    TB = B // 128
    tb_per_w = TB // _NW
    n_sg = S // _SG
    n_ct = dim // 8

    ids_t = jnp.transpose(token_ids)

    mesh = plsc.VectorSubcoreMesh(core_axis_name="c", subcore_axis_name="s")

    @functools.partial(
        pl.kernel,
        mesh=mesh,
        out_type=jax.ShapeDtypeStruct((S, n_ct, TB, 8, 128), W.dtype),
        scratch_types=[
            pltpu.VMEM((S, 128), jnp.int32),
            pltpu.VMEM((_SG, 128, 32), W.dtype),
            pltpu.VMEM((_SG, 32, _OP), W.dtype),
            pltpu.SemaphoreType.DMA,
            pltpu.SemaphoreType.DMA,
        ],
        compiler_params=pltpu.CompilerParams(
            use_tc_tiling_on_sc=False, needs_layout_passes=False
        ),
    )
    def gather_kernel(
        table_hbm, idx_hbm, out_hbm, idx_v, g_v, o_v, gsem, osem
    ):
        wid = lax.axis_index("s") * _NC + lax.axis_index("c")

        iota = jax.lax.broadcasted_iota(jnp.int32, (_L,), 0)
        cvecs = [iota + k * _L for k in range(dim // _L)]

        @pl.loop(0, tb_per_w)
        def _(t):
            tb = wid * tb_per_w + t
            pltpu.sync_copy(idx_hbm.at[:, pl.ds(tb * 128, 128)], idx_v)

            @pl.loop(0, n_sg)
            def _(sg):
                s0 = sg * _SG
                for j in range(_SG):
                    pltpu.async_copy(
                        table_hbm.at[idx_v.at[s0 + j]], g_v.at[j], gsem
                    )
                for j in range(_SG):
                    pltpu.make_async_copy(
                        table_hbm.at[idx_v.at[s0 + j]], g_v.at[j], gsem
                    ).wait()
                    jv = jnp.full((_L,), j, jnp.int32)

                    @pl.loop(0, 128 // _L)
                    def _(blb):
                        for bi in range(_L):
                            bl = blb * _L + bi
                            blv = jnp.full((_L,), bl, jnp.int32)
                            for k in range(dim // _L):
                                vals = g_v[j, bl, pl.ds(k * _L, _L)]
                                plsc.store_scatter(
                                    o_v, [jv, cvecs[k], blv], vals
                                )
                    for tc in range(n_ct):
                        pltpu.async_copy(
                            o_v.at[j, pl.ds(tc * 8, 8), pl.ds(0, 128)],
                            out_hbm.at[s0 + j, tc, tb],
                            osem,
                        )
                for j in range(_SG):
                    for tc in range(n_ct):
                        pltpu.make_async_copy(
                            o_v.at[j, pl.ds(tc * 8, 8), pl.ds(0, 128)],
                            out_hbm.at[s0 + j, tc, tb],
                            osem,
                        ).wait()

    out_a = gather_kernel(W, ids_t)
    t = jnp.transpose(out_a, (2, 4, 0, 1, 3))
    return t.reshape(B, S, dim)


# speedup vs baseline: 1.5693x; 1.0142x over previous
import functools

import jax
import jax.numpy as jnp
from jax import lax
from jax.experimental import pallas as pl
from jax.experimental.pallas import tpu as pltpu
from jax.experimental.pallas import tpu_sc as plsc

_NC = 2
_NS = 16
_NW = _NC * _NS
_L = 16
_SG = 5
_OP = 129


def kernel(token_ids, W):
    B, S = token_ids.shape
    dim = W.shape[1]
    TB = B // 128
    tb_per_w = TB // _NW
    n_sg = S // _SG
    n_ct = dim // 8

    ids_t = jnp.transpose(token_ids)

    mesh = plsc.VectorSubcoreMesh(core_axis_name="c", subcore_axis_name="s")

    @functools.partial(
        pl.kernel,
        mesh=mesh,
        out_type=jax.ShapeDtypeStruct((S, n_ct, TB, 8, 128), W.dtype),
        scratch_types=[
            pltpu.VMEM((S, 128), jnp.int32),
            pltpu.VMEM((_SG, 128, 32), W.dtype),
            pltpu.VMEM((_SG, 128, 32), W.dtype),
            pltpu.VMEM((_SG, 32, _OP), W.dtype),
            pltpu.SemaphoreType.DMA,
            pltpu.SemaphoreType.DMA,
            pltpu.SemaphoreType.DMA,
        ],
        compiler_params=pltpu.CompilerParams(
            use_tc_tiling_on_sc=False, needs_layout_passes=False
        ),
    )
    def gather_kernel(
        table_hbm, idx_hbm, out_hbm, idx_v, g0_v, g1_v, o_v, g0sem, g1sem, osem
    ):
        wid = lax.axis_index("s") * _NC + lax.axis_index("c")

        iota = jax.lax.broadcasted_iota(jnp.int32, (_L,), 0)
        cvecs = [iota + k * _L for k in range(dim // _L)]

        def fire(sg, g_v, gsem):
            s0 = sg * _SG
            for j in range(_SG):
                pltpu.async_copy(
                    table_hbm.at[idx_v.at[s0 + j]], g_v.at[j], gsem
                )

        def process(sg, tb, g_v, gsem):
            s0 = sg * _SG
            for j in range(_SG):
                pltpu.make_async_copy(
                    table_hbm.at[idx_v.at[s0 + j]], g_v.at[j], gsem
                ).wait()
                jv = jnp.full((_L,), j, jnp.int32)

                @pl.loop(0, 128 // _L)
                def _(blb):
                    for bi in range(_L):
                        bl = blb * _L + bi
                        blv = jnp.full((_L,), bl, jnp.int32)
                        for k in range(dim // _L):
                            vals = g_v[j, bl, pl.ds(k * _L, _L)]
                            plsc.store_scatter(
                                o_v, [jv, cvecs[k], blv], vals
                            )
                for tc in range(n_ct):
                    pltpu.async_copy(
                        o_v.at[j, pl.ds(tc * 8, 8), pl.ds(0, 128)],
                        out_hbm.at[s0 + j, tc, tb],
                        osem,
                    )
            for j in range(_SG):
                for tc in range(n_ct):
                    pltpu.make_async_copy(
                        o_v.at[j, pl.ds(tc * 8, 8), pl.ds(0, 128)],
                        out_hbm.at[s0 + j, tc, tb],
                        osem,
                    ).wait()

        @pl.loop(0, tb_per_w)
        def _(t):
            tb = wid * tb_per_w + t
            pltpu.sync_copy(idx_hbm.at[:, pl.ds(tb * 128, 128)], idx_v)
            fire(0, g0_v, g0sem)

            @pl.loop(0, n_sg // 2)
            def _(p):
                sg0 = p * 2
                fire(sg0 + 1, g1_v, g1sem)
                process(sg0, tb, g0_v, g0sem)

                @pl.when(p + 1 < n_sg // 2)
                def _():
                    fire(sg0 + 2, g0_v, g0sem)

                process(sg0 + 1, tb, g1_v, g1sem)

    out_a = gather_kernel(W, ids_t)
    t = jnp.transpose(out_a, (2, 4, 0, 1, 3))
    return t.reshape(B, S, dim)
